# Initial kernel scaffold; baseline (speedup 1.0000x reference)
import jax
import jax.numpy as jnp
from jax.experimental import pallas as pl


def kernel(pokemon_ids, ability_ids, item_ids, move_ids, pokemon_static, pokemon_learn, ability_static, ability_learn, item_static, item_learn, move_static, move_learn):
    raise NotImplementedError("write your pallas kernel here")


# speedup vs baseline: 7.8759x; 7.8759x over previous
import functools

import jax
import jax.numpy as jnp
from jax import lax
from jax.experimental import pallas as pl
from jax.experimental.pallas import tpu as pltpu
from jax.experimental.pallas import tpu_sc as plsc

N = 196608
NUM_WORKERS = 32
R = 64
BLOCKS_PER_WORKER = N // (NUM_WORKERS * R)
NBLK = N // R
OUT_W = 1111

SEG_C = [0, 291, 342, 393, 444, 495, 649, 803, 957]
SEG_W = [291, 51, 51, 51, 51, 154, 154, 154, 154]
SEG_P = [c % 8 for c in SEG_C]
SEG_A = [c - p for c, p in zip(SEG_C, SEG_P)]
SEG_WW = [SEG_A[k + 1] - SEG_A[k] for k in range(8)] + [OUT_W - SEG_A[8]]
SEG_BW = []
for k in range(9):
    need = SEG_P[k] + SEG_W[k]
    if k < 8:
        need = max(need, SEG_A[k + 1] - SEG_A[k] + 16)
    if need % 8 and k < 8:
        need += 8 - need % 8
    SEG_BW.append(need)


def _sc_body(*refs):
    tabs = refs[0:9]
    idxs = refs[9]
    out = refs[10]
    idx_s = refs[11]
    spans = refs[12:21]
    g8 = refs[21]
    gsem = refs[22]
    wsem = refs[23]

    wid = lax.axis_index("s") * 2 + lax.axis_index("c")
    lane = lax.iota(jnp.int32, 16)

    def block(g, carry):
        blk = wid * BLOCKS_PER_WORKER + g
        row0 = blk * R
        pltpu.sync_copy(idxs.at[blk], idx_s)
        cps = [pltpu.async_copy(tabs[k].at[idx_s.at[k]],
                                spans[k].at[pl.ds(0, R)], gsem)
               for k in range(8)]
        cps.append(pltpu.async_copy(tabs[8].at[idx_s.at[8]],
                                    g8.at[pl.ds(0, R)], gsem))
        for cp in cps:
            cp.wait()

        def fix(r, c):
            for col in range(0, 144, 16):
                spans[8][r, pl.ds(col, 16)] = g8[r, pl.ds(col, 16)]
            spans[8][r, pl.ds(143, 16)] = g8[r, pl.ds(143, 16)]
            for k in range(1, 9):
                off = SEG_A[k] - SEG_A[k - 1]
                prev = spans[k - 1][r, pl.ds(off, 16)]
                cur = spans[k][r, pl.ds(0, 16)]
                spans[k][r, pl.ds(0, 16)] = jnp.where(lane < SEG_P[k], prev, cur)
            return c
        lax.fori_loop(0, R, fix, 0)

        wps = [pltpu.async_copy(
                   spans[k].at[pl.ds(0, R), pl.ds(0, SEG_WW[k])],
                   out.at[pl.ds(row0, R), pl.ds(SEG_A[k], SEG_WW[k])], wsem)
               for k in range(9)]
        for wp in wps:
            wp.wait()
        return carry

    lax.fori_loop(0, BLOCKS_PER_WORKER, block, 0)


@jax.jit
def _sc_encode(tabs, idxs):
    mesh = plsc.VectorSubcoreMesh(core_axis_name="c", subcore_axis_name="s")
    kern = functools.partial(
        pl.kernel,
        mesh=mesh,
        out_type=jax.ShapeDtypeStruct((N, OUT_W), jnp.float32),
        scratch_types=[pltpu.VMEM((9, R), jnp.int32)]
        + [pltpu.VMEM((R, SEG_BW[k]), jnp.float32) for k in range(9)]
        + [pltpu.VMEM((R, 160), jnp.float32)]
        + [pltpu.SemaphoreType.DMA, pltpu.SemaphoreType.DMA],
        compiler_params=pltpu.CompilerParams(use_tc_tiling_on_sc=False),
    )(_sc_body)
    return kern(*tabs, idxs)


def _pad_tab(tab, k):
    tw = 160 if k == 8 else SEG_BW[k]
    left = SEG_P[k]
    right = tw - SEG_P[k] - SEG_W[k]
    return jnp.pad(tab, ((0, 0), (left, right)))


def kernel(pokemon_ids, ability_ids, item_ids, move_ids,
           pokemon_static, pokemon_learn, ability_static, ability_learn,
           item_static, item_learn, move_static, move_learn):
    pk_tab = jnp.concatenate([pokemon_static, pokemon_learn], axis=1)
    ab_tab = jnp.concatenate([ability_static, ability_learn], axis=1)
    it_tab = jnp.concatenate([item_static, item_learn], axis=1)
    mv_tab = jnp.concatenate([move_static, move_learn], axis=1)
    src = [pk_tab, ab_tab, ab_tab, ab_tab, it_tab, mv_tab, mv_tab, mv_tab, mv_tab]
    tabs = [_pad_tab(src[k], k) for k in range(9)]
    idx_all = jnp.concatenate(
        [pokemon_ids[:, None], ability_ids, item_ids, move_ids], axis=1)
    idxs = idx_all.T.reshape(9, NBLK, R).transpose(1, 0, 2)
    return _sc_encode(tabs, idxs)


# speedup vs baseline: 9.1043x; 1.1560x over previous
import functools

import jax
import jax.numpy as jnp
from jax import lax
from jax.experimental import pallas as pl
from jax.experimental.pallas import tpu as pltpu
from jax.experimental.pallas import tpu_sc as plsc

N = 196608
NUM_WORKERS = 32
R = 32
G_BLOCKS = N // (NUM_WORKERS * R)
NBLK = N // R
OUT_W = 1111

SEG_C = [0, 291, 342, 393, 444, 495, 649, 803, 957]
SEG_W = [291, 51, 51, 51, 51, 154, 154, 154, 154]
SEG_P = [c % 8 for c in SEG_C]
SEG_A = [c - p for c, p in zip(SEG_C, SEG_P)]
SEG_WW = [SEG_A[k + 1] - SEG_A[k] for k in range(8)] + [OUT_W - SEG_A[8]]
SEG_BW = []
for k in range(9):
    need = SEG_P[k] + SEG_W[k]
    if k < 8:
        need = max(need, SEG_A[k + 1] - SEG_A[k] + 16)
    if need % 8 and k < 8:
        need += 8 - need % 8
    SEG_BW.append(need)
G8_W = 160
SEG_TW = SEG_BW[:8] + [G8_W]


def _sc_body(*refs):
    tabs = refs[0:9]
    idxs = refs[9]
    out = refs[10]
    idx_s = refs[11]
    spans = refs[12:21]
    g8 = refs[21]
    gsem = refs[22:24]
    wsem = refs[24:26]

    wid = lax.axis_index("s") * 2 + lax.axis_index("c")
    base_blk = wid * G_BLOCKS
    lane = lax.iota(jnp.int32, 16)

    def issue_gathers(s, g):
        blk = base_blk + g
        pltpu.sync_copy(idxs.at[blk], idx_s.at[s])
        for k in range(8):
            pltpu.async_copy(tabs[k].at[idx_s.at[s, k]], spans[k].at[s], gsem[s])
        pltpu.async_copy(tabs[8].at[idx_s.at[s, 8]], g8.at[s], gsem[s])

    def wait_gathers(s):
        for k in range(8):
            pltpu.make_async_copy(tabs[k].at[pl.ds(0, R)], spans[k].at[s],
                                  gsem[s]).wait()
        pltpu.make_async_copy(tabs[8].at[pl.ds(0, R)], g8.at[s], gsem[s]).wait()

    def fix(s):
        def fr(r, c):
            for col in range(0, 144, 16):
                spans[8][s, r, pl.ds(col, 16)] = g8[s, r, pl.ds(col, 16)]
            spans[8][s, r, pl.ds(143, 16)] = g8[s, r, pl.ds(143, 16)]
            for k in range(1, 9):
                off = SEG_A[k] - SEG_A[k - 1]
                prev = spans[k - 1][s, r, pl.ds(off, 16)]
                cur = spans[k][s, r, pl.ds(0, 16)]
                spans[k][s, r, pl.ds(0, 16)] = jnp.where(lane < SEG_P[k], prev, cur)
            return c
        lax.fori_loop(0, R, fr, 0)

    def issue_writes(s, g):
        row0 = (base_blk + g) * R
        for k in range(9):
            pltpu.async_copy(
                spans[k].at[s, pl.ds(0, R), pl.ds(0, SEG_WW[k])],
                out.at[pl.ds(row0, R), pl.ds(SEG_A[k], SEG_WW[k])], wsem[s])

    def wait_writes(s):
        for k in range(9):
            pltpu.make_async_copy(
                spans[k].at[s, pl.ds(0, R), pl.ds(0, SEG_WW[k])],
                out.at[pl.ds(0, R), pl.ds(SEG_A[k], SEG_WW[k])], wsem[s]).wait()

    issue_gathers(0, 0)

    def pair(h, c):
        for s in (0, 1):
            g = 2 * h + s
            o = 1 - s

            @pl.when(g + 1 < G_BLOCKS)
            def _():
                if s == 0:
                    @pl.when(h >= 1)
                    def _():
                        wait_writes(o)
                else:
                    wait_writes(o)
                issue_gathers(o, g + 1)

            wait_gathers(s)
            fix(s)
            issue_writes(s, g)
        return c

    lax.fori_loop(0, G_BLOCKS // 2, pair, 0)
    wait_writes(0)
    wait_writes(1)


@jax.jit
def _sc_encode(tabs, idxs):
    mesh = plsc.VectorSubcoreMesh(core_axis_name="c", subcore_axis_name="s")
    kern = functools.partial(
        pl.kernel,
        mesh=mesh,
        out_type=jax.ShapeDtypeStruct((N, OUT_W), jnp.float32),
        scratch_types=[pltpu.VMEM((2, 9, R), jnp.int32)]
        + [pltpu.VMEM((2, R, SEG_BW[k]), jnp.float32) for k in range(9)]
        + [pltpu.VMEM((2, R, G8_W), jnp.float32)]
        + [pltpu.SemaphoreType.DMA] * 4,
        compiler_params=pltpu.CompilerParams(use_tc_tiling_on_sc=False),
    )(_sc_body)
    return kern(*tabs, idxs)


def _pad_tab(tab, k):
    left = SEG_P[k]
    right = SEG_TW[k] - SEG_P[k] - SEG_W[k]
    return jnp.pad(tab, ((0, 0), (left, right)))


def kernel(pokemon_ids, ability_ids, item_ids, move_ids,
           pokemon_static, pokemon_learn, ability_static, ability_learn,
           item_static, item_learn, move_static, move_learn):
    pk_tab = jnp.concatenate([pokemon_static, pokemon_learn], axis=1)
    ab_tab = jnp.concatenate([ability_static, ability_learn], axis=1)
    it_tab = jnp.concatenate([item_static, item_learn], axis=1)
    mv_tab = jnp.concatenate([move_static, move_learn], axis=1)
    src = [pk_tab, ab_tab, ab_tab, ab_tab, it_tab, mv_tab, mv_tab, mv_tab, mv_tab]
    tabs = [_pad_tab(src[k], k) for k in range(9)]
    idx_all = jnp.concatenate(
        [pokemon_ids[:, None], ability_ids, item_ids, move_ids], axis=1)
    idxs = idx_all.T.reshape(9, NBLK, R).transpose(1, 0, 2)
    return _sc_encode(tabs, idxs)
